# baseline (speedup 1.0000x reference)
import functools

import jax
import jax.numpy as jnp
from jax import lax
from jax.experimental import pallas as pl
from jax.experimental.pallas import tpu as pltpu
from jax.experimental.pallas import tpu_sc as plsc

H = 256
EPS = 1e-08

NC = 2
NS = 16
LANES = 16
NCHUNK = 16
CW = 16
B = 80


def _mlp_a_body(s_ref, w1_ref, b1_ref, w2_ref, b2_ref, o_ref):
    h = jnp.dot(s_ref[...], w1_ref[...], preferred_element_type=jnp.float32)
    h = h + b1_ref[...]
    h = h * jax.nn.sigmoid(h)
    o = jnp.dot(h, w2_ref[...], preferred_element_type=jnp.float32)
    o_ref[...] = o + b2_ref[...]


def _interaction(s2, Wi1, bi1, Wi2, bi2):
    n = s2.shape[0]
    r = 2000
    grid = n // r
    return pl.pallas_call(
        _mlp_a_body,
        grid=(grid,),
        in_specs=[
            pl.BlockSpec((r, H), lambda i: (i, 0)),
            pl.BlockSpec((H, H), lambda i: (0, 0)),
            pl.BlockSpec((1, H), lambda i: (0, 0)),
            pl.BlockSpec((H, 3 * H), lambda i: (0, 0)),
            pl.BlockSpec((1, 3 * H), lambda i: (0, 0)),
        ],
        out_specs=pl.BlockSpec((r, 3 * H), lambda i: (i, 0)),
        out_shape=jax.ShapeDtypeStruct((n, 3 * H), jnp.float32),
    )(s2, Wi1, bi1.reshape(1, H), Wi2, bi2.reshape(1, 3 * H))


def _edge_phase(x8, v8, wij, dir_flat, senders, receivers):
    n8 = x8.shape[0]
    n = n8 // NCHUNK
    e = senders.shape[0]
    nblk_per_tile = e // B // NS
    nquad = (nblk_per_tile - 1) // 4
    rows_per_tile = n // NS

    mesh = plsc.VectorSubcoreMesh(
        core_axis_name="c", subcore_axis_name="s",
        num_cores=NC, num_subcores=NS)

    DEPTH = 4
    buf_set = [
        pltpu.VMEM((B,), jnp.int32),
        pltpu.VMEM((B,), jnp.int32),
        pltpu.VMEM((B,), jnp.int32),
        pltpu.VMEM((B, 3 * CW), jnp.float32),
        pltpu.VMEM((B, 3 * CW), jnp.float32),
        pltpu.VMEM((B, CW), jnp.float32),
        pltpu.VMEM((B, CW), jnp.float32),
        pltpu.VMEM((B, CW), jnp.float32),
        pltpu.VMEM((B * 3 + LANES,), jnp.float32),
        pltpu.VMEM((B, 4 * CW), jnp.float32),
    ]

    @functools.partial(
        pl.kernel,
        out_type=(
            jax.ShapeDtypeStruct((n, NCHUNK, CW), jnp.float32),
            jax.ShapeDtypeStruct((n, 3, NCHUNK, CW), jnp.float32),
        ),
        mesh=mesh,
        scratch_types=[
            pltpu.VMEM_SHARED((n, 4 * CW), jnp.float32),
            pltpu.VMEM((25, 4 * CW), jnp.float32),
            pltpu.SemaphoreType.DMA((DEPTH,)),
        ] + buf_set * DEPTH,
        compiler_params=pltpu.CompilerParams(use_tc_tiling_on_sc=False),
    )
    def ek(x_hbm, v_hbm, w_hbm, dir_hbm, snd_hbm, rcv_hbm, ds_out, dv_out,
           accum, zbuf, sems, *bufs):
        cid = lax.axis_index("c")
        tid = lax.axis_index("s")
        nb = len(buf_set)
        pbufs = tuple(bufs[i * nb:(i + 1) * nb] for i in range(DEPTH))

        nz = (4 * CW) // LANES
        def zfill(i, _):
            zbuf[i // nz, pl.ds((i % nz) * LANES, LANES)] = jnp.zeros(
                (LANES,), jnp.float32)
            return 0
        lax.fori_loop(0, 25 * nz, zfill, 0)

        n0 = tid * rows_per_tile

        for ch_l in range(NCHUNK // NC):
            ch = cid * (NCHUNK // NC) + ch_l

            for z in range(rows_per_tile // 25):
                pltpu.sync_copy(
                    zbuf, accum.at[pl.ds(n0 + z * 25, 25), :])
            plsc.subcore_barrier()

            def issue(bi, p):
                (sidx, ridx, gx, xblk, vblk,
                 wds, wd1, wd2, dirb, orows) = pbufs[p]
                sem = sems.at[p]
                e0 = (bi * NS + tid) * B
                pltpu.sync_copy(rcv_hbm.at[pl.ds(e0, B)], ridx)

                def idx_body(k, _):
                    sl = pl.ds(k * LANES, LANES)
                    gx[sl] = ridx[sl] * NCHUNK + ch
                    return 0
                lax.fori_loop(0, B // LANES, idx_body, 0)

                return (
                    pltpu.async_copy(x_hbm.at[gx], xblk, sem),
                    pltpu.async_copy(v_hbm.at[gx], vblk, sem),
                    pltpu.async_copy(w_hbm.at[pl.ds(e0, B), ch, :], wds,
                                     sem),
                    pltpu.async_copy(
                        w_hbm.at[pl.ds(e0, B), NCHUNK + ch, :], wd1, sem),
                    pltpu.async_copy(
                        w_hbm.at[pl.ds(e0, B), 2 * NCHUNK + ch, :], wd2,
                        sem),
                    pltpu.async_copy(snd_hbm.at[pl.ds(e0, B)], sidx, sem),
                    pltpu.async_copy(dir_hbm.at[pl.ds(e0 * 3, B * 3)],
                                     dirb.at[pl.ds(0, B * 3)], sem),
                )

            def compute(descs, p):
                (sidx, ridx, gx, xblk, vblk,
                 wds, wd1, wd2, dirb, orows) = pbufs[p]
                for d in descs:
                    d.wait()

                def e_body(ei, _):
                    d3 = dirb[pl.ds(ei * 3, LANES)]
                    dd0 = d3[0]
                    dd1 = d3[1]
                    dd2 = d3[2]
                    sl = pl.ds(0, LANES)
                    sl1 = pl.ds(CW, LANES)
                    sl2 = pl.ds(2 * CW, LANES)
                    a1 = xblk[ei, sl1] * wd1[ei, sl]
                    a2 = xblk[ei, sl2] * wd2[ei, sl]
                    orows[ei, pl.ds(0, LANES)] = xblk[ei, sl] * wds[ei, sl]
                    orows[ei, pl.ds(CW, LANES)] = a1 * dd0 + a2 * vblk[ei, sl]
                    orows[ei, pl.ds(2 * CW, LANES)] = (
                        a1 * dd1 + a2 * vblk[ei, sl1])
                    orows[ei, pl.ds(3 * CW, LANES)] = (
                        a1 * dd2 + a2 * vblk[ei, sl2])
                    return 0
                lax.fori_loop(0, B, e_body, 0)

                pltpu.sync_copy(orows, accum.at[sidx], add=True)

            def quad_body(k, _):
                descs = [issue(4 * k + p, p) for p in range(DEPTH)]
                for p in range(DEPTH):
                    compute(descs[p], p)
                return 0
            lax.fori_loop(0, nquad, quad_body, 0)
            dt = issue(jnp.int32(nblk_per_tile - 1), 0)
            compute(dt, 0)
            plsc.subcore_barrier()

            pltpu.sync_copy(
                accum.at[pl.ds(n0, rows_per_tile), pl.ds(0, CW)],
                ds_out.at[pl.ds(n0, rows_per_tile), ch, :])
            for d in range(3):
                pltpu.sync_copy(
                    accum.at[pl.ds(n0, rows_per_tile),
                             pl.ds((d + 1) * CW, CW)],
                    dv_out.at[pl.ds(n0, rows_per_tile), d, ch, :])

    return ek(x8, v8, wij, dir_flat, senders, receivers)


def _mix_body(s_ref, v_ref, ds_ref, dv_ref, wv_ref, bv_ref,
              wm1a_ref, wm1b_ref, bm1_ref, wm2_ref, bm2_ref,
              so_ref, vo_ref):
    s1 = s_ref[...] + ds_ref[...]
    u0 = v_ref[:, 0, :] + dv_ref[:, 0, :]
    u1 = v_ref[:, 1, :] + dv_ref[:, 1, :]
    u2 = v_ref[:, 2, :] + dv_ref[:, 2, :]
    wv = wv_ref[...]
    bv = bv_ref[...]
    m0 = jnp.dot(u0, wv, preferred_element_type=jnp.float32) + bv
    m1 = jnp.dot(u1, wv, preferred_element_type=jnp.float32) + bv
    m2 = jnp.dot(u2, wv, preferred_element_type=jnp.float32) + bv
    l0, r0 = m0[:, :H], m0[:, H:]
    l1, r1 = m1[:, :H], m1[:, H:]
    l2, r2 = m2[:, :H], m2[:, H:]
    vnorm = jnp.sqrt(l0 * l0 + l1 * l1 + l2 * l2 + EPS)
    h = (jnp.dot(s1, wm1a_ref[...], preferred_element_type=jnp.float32)
         + jnp.dot(vnorm, wm1b_ref[...], preferred_element_type=jnp.float32)
         + bm1_ref[...])
    h = h * jax.nn.sigmoid(h)
    mix = jnp.dot(h, wm2_ref[...], preferred_element_type=jnp.float32)
    mix = mix + bm2_ref[...]
    ds2 = mix[:, :H]
    dvu = mix[:, H:2 * H]
    dsv = mix[:, 2 * H:]
    dot_lr = l0 * r0 + l1 * r1 + l2 * r2
    so_ref[...] = s1 + ds2 + dsv * dot_lr
    vo_ref[:, 0, :] = u0 + dvu * r0
    vo_ref[:, 1, :] = u1 + dvu * r1
    vo_ref[:, 2, :] = u2 + dvu * r2


def _mixing(s2, v, ds, dv, Wv, bv, Wm1, bm1, Wm2, bm2):
    n = s2.shape[0]
    r = 1000
    grid = n // r
    return pl.pallas_call(
        _mix_body,
        grid=(grid,),
        in_specs=[
            pl.BlockSpec((r, H), lambda i: (i, 0)),
            pl.BlockSpec((r, 3, H), lambda i: (i, 0, 0)),
            pl.BlockSpec((r, H), lambda i: (i, 0)),
            pl.BlockSpec((r, 3, H), lambda i: (i, 0, 0)),
            pl.BlockSpec((H, 2 * H), lambda i: (0, 0)),
            pl.BlockSpec((1, 2 * H), lambda i: (0, 0)),
            pl.BlockSpec((H, H), lambda i: (0, 0)),
            pl.BlockSpec((H, H), lambda i: (0, 0)),
            pl.BlockSpec((1, H), lambda i: (0, 0)),
            pl.BlockSpec((H, 3 * H), lambda i: (0, 0)),
            pl.BlockSpec((1, 3 * H), lambda i: (0, 0)),
        ],
        out_specs=[
            pl.BlockSpec((r, H), lambda i: (i, 0)),
            pl.BlockSpec((r, 3, H), lambda i: (i, 0, 0)),
        ],
        out_shape=[
            jax.ShapeDtypeStruct((n, H), jnp.float32),
            jax.ShapeDtypeStruct((n, 3, H), jnp.float32),
        ],
    )(s2, v, ds, dv, Wv, bv.reshape(1, 2 * H), Wm1[:H], Wm1[H:],
      bm1.reshape(1, H), Wm2, bm2.reshape(1, 3 * H))


def kernel(s, v, dir_ij, Wij, senders, receivers,
           Wi1, bi1, Wi2, bi2, Wv, bv, Wm1, bm1, Wm2, bm2):
    n = s.shape[0]
    e = senders.shape[0]
    s2 = s.reshape(n, H)
    wi2p = Wi2.reshape(H, 3, NCHUNK, CW).transpose(0, 2, 1, 3).reshape(
        H, 3 * H)
    bi2p = bi2.reshape(3, NCHUNK, CW).transpose(1, 0, 2).reshape(3 * H)
    x = _interaction(s2, Wi1, bi1, wi2p, bi2p)
    x8 = x.reshape(n * NCHUNK, 3 * CW)
    v8 = v.reshape(n, 3, NCHUNK, CW).transpose(0, 2, 1, 3).reshape(
        n * NCHUNK, 3 * CW)
    wij = Wij.reshape(e, 3 * NCHUNK, CW)
    ds8, dv8 = _edge_phase(x8, v8, wij, dir_ij.reshape(e * 3),
                           senders, receivers)
    ds = ds8.reshape(n, H)
    dv = dv8.reshape(n, 3, H)
    so, vo = _mixing(s2, v, ds, dv, Wv, bv, Wm1, bm1, Wm2, bm2)
    return (so.reshape(n, 1, H), vo)


# speedup vs baseline: 1.2385x; 1.2385x over previous
import functools

import jax
import jax.numpy as jnp
from jax import lax
from jax.experimental import pallas as pl
from jax.experimental.pallas import tpu as pltpu
from jax.experimental.pallas import tpu_sc as plsc

H = 256
EPS = 1e-08

NC = 2
NS = 16
LANES = 16
NCHUNK = 8
CW = 32
B = 80


def _mlp_a_body(s_ref, w1_ref, b1_ref, w2_ref, b2_ref, o_ref):
    h = jnp.dot(s_ref[...], w1_ref[...], preferred_element_type=jnp.float32)
    h = h + b1_ref[...]
    h = h * jax.nn.sigmoid(h)
    o = jnp.dot(h, w2_ref[...], preferred_element_type=jnp.float32)
    o_ref[...] = o + b2_ref[...]


def _interaction(s2, Wi1, bi1, Wi2, bi2):
    n = s2.shape[0]
    r = 2000
    grid = n // r
    return pl.pallas_call(
        _mlp_a_body,
        grid=(grid,),
        in_specs=[
            pl.BlockSpec((r, H), lambda i: (i, 0)),
            pl.BlockSpec((H, H), lambda i: (0, 0)),
            pl.BlockSpec((1, H), lambda i: (0, 0)),
            pl.BlockSpec((H, 3 * H), lambda i: (0, 0)),
            pl.BlockSpec((1, 3 * H), lambda i: (0, 0)),
        ],
        out_specs=pl.BlockSpec((r, 3 * H), lambda i: (i, 0)),
        out_shape=jax.ShapeDtypeStruct((n, 3 * H), jnp.float32),
    )(s2, Wi1, bi1.reshape(1, H), Wi2, bi2.reshape(1, 3 * H))


def _edge_phase(x8, v8, wij, dir_flat, sr2, n):
    e = sr2.shape[1]
    nblk_per_tile = e // B // NS
    npair = (nblk_per_tile - 1) // 2
    rows_per_tile = n // NS

    mesh = plsc.VectorSubcoreMesh(
        core_axis_name="c", subcore_axis_name="s",
        num_cores=NC, num_subcores=NS)

    DEPTH = 2
    buf_set = [
        pltpu.VMEM((2, B), jnp.int32),
        pltpu.VMEM((B,), jnp.int32),
        pltpu.VMEM((B, 3 * CW), jnp.float32),
        pltpu.VMEM((B, 3 * CW), jnp.float32),
        pltpu.VMEM((B * 3 + LANES,), jnp.float32),
    ]

    @functools.partial(
        pl.kernel,
        out_type=(
            jax.ShapeDtypeStruct((n, NCHUNK, CW), jnp.float32),
            jax.ShapeDtypeStruct((n, 3, NCHUNK, CW), jnp.float32),
        ),
        mesh=mesh,
        scratch_types=[
            pltpu.VMEM_SHARED((n, 4 * CW), jnp.float32),
            pltpu.VMEM((5, 4 * CW), jnp.float32),
            pltpu.VMEM((B, 3, CW), jnp.float32),
            pltpu.VMEM((B, 4 * CW), jnp.float32),
            pltpu.SemaphoreType.DMA((DEPTH,)),
        ] + buf_set * DEPTH,
        compiler_params=pltpu.CompilerParams(use_tc_tiling_on_sc=False),
    )
    def ek(x_hbm, v_hbm, w_hbm, dir_hbm, sr_hbm, ds_out, dv_out,
           accum, zbuf, wbuf, orows, sems, *bufs):
        cid = lax.axis_index("c")
        tid = lax.axis_index("s")
        nb = len(buf_set)
        pbufs = tuple(bufs[i * nb:(i + 1) * nb] for i in range(DEPTH))

        nz = (4 * CW) // LANES
        def zfill(i, _):
            zbuf[i // nz, pl.ds((i % nz) * LANES, LANES)] = jnp.zeros(
                (LANES,), jnp.float32)
            return 0
        lax.fori_loop(0, 5 * nz, zfill, 0)

        n0 = tid * rows_per_tile

        for ch_l in range(NCHUNK // NC):
            ch = cid * (NCHUNK // NC) + ch_l

            for z in range(rows_per_tile // 5):
                pltpu.sync_copy(
                    zbuf, accum.at[pl.ds(n0 + z * 5, 5), :])
            plsc.subcore_barrier()

            def issue(bi, p):
                (srb, gx, xblk, vblk, dirb) = pbufs[p]
                sem = sems.at[p]
                e0 = (bi * NS + tid) * B
                pltpu.sync_copy(sr_hbm.at[:, pl.ds(e0, B)], srb)

                def idx_body(k, _):
                    sl = pl.ds(k * LANES, LANES)
                    gx[sl] = srb[0, sl] * NCHUNK + ch
                    return 0
                lax.fori_loop(0, B // LANES, idx_body, 0)

                return (
                    pltpu.async_copy(x_hbm.at[gx], xblk, sem),
                    pltpu.async_copy(v_hbm.at[gx], vblk, sem),
                    pltpu.async_copy(dir_hbm.at[pl.ds(e0 * 3, B * 3)],
                                     dirb.at[pl.ds(0, B * 3)], sem),
                )

            def compute(descs, bi, p):
                (srb, gx, xblk, vblk, dirb) = pbufs[p]
                e0 = (bi * NS + tid) * B
                pltpu.sync_copy(w_hbm.at[pl.ds(e0, B), :, ch, :], wbuf)
                for d in descs:
                    d.wait()

                def e_body(ei, _):
                    d3 = dirb[pl.ds(ei * 3, LANES)]
                    dd0 = d3[0]
                    dd1 = d3[1]
                    dd2 = d3[2]
                    for j in range(CW // LANES):
                        sl = pl.ds(j * LANES, LANES)
                        sl1 = pl.ds(CW + j * LANES, LANES)
                        sl2 = pl.ds(2 * CW + j * LANES, LANES)
                        a1 = xblk[ei, sl1] * wbuf[ei, 1, sl]
                        a2 = xblk[ei, sl2] * wbuf[ei, 2, sl]
                        orows[ei, pl.ds(j * LANES, LANES)] = (
                            xblk[ei, sl] * wbuf[ei, 0, sl])
                        orows[ei, pl.ds(CW + j * LANES, LANES)] = (
                            a1 * dd0 + a2 * vblk[ei, sl])
                        orows[ei, pl.ds(2 * CW + j * LANES, LANES)] = (
                            a1 * dd1 + a2 * vblk[ei, sl1])
                        orows[ei, pl.ds(3 * CW + j * LANES, LANES)] = (
                            a1 * dd2 + a2 * vblk[ei, sl2])
                    return 0
                lax.fori_loop(0, B, e_body, 0)

                pltpu.sync_copy(orows, accum.at[srb.at[1]], add=True)

            def pair_body(k, _):
                d0 = issue(2 * k, 0)
                d1 = issue(2 * k + 1, 1)
                compute(d0, 2 * k, 0)
                compute(d1, 2 * k + 1, 1)
                return 0
            lax.fori_loop(0, npair, pair_body, 0)
            dt = issue(jnp.int32(nblk_per_tile - 1), 0)
            compute(dt, jnp.int32(nblk_per_tile - 1), 0)
            plsc.subcore_barrier()

            pltpu.sync_copy(
                accum.at[pl.ds(n0, rows_per_tile), pl.ds(0, CW)],
                ds_out.at[pl.ds(n0, rows_per_tile), ch, :])
            for d in range(3):
                pltpu.sync_copy(
                    accum.at[pl.ds(n0, rows_per_tile),
                             pl.ds((d + 1) * CW, CW)],
                    dv_out.at[pl.ds(n0, rows_per_tile), d, ch, :])

    return ek(x8, v8, wij, dir_flat, sr2)


def _mix_body(s_ref, v_ref, ds_ref, dv_ref, wv_ref, bv_ref,
              wm1a_ref, wm1b_ref, bm1_ref, wm2_ref, bm2_ref,
              so_ref, vo_ref):
    s1 = s_ref[...] + ds_ref[...]
    u0 = v_ref[:, 0, :] + dv_ref[:, 0, :]
    u1 = v_ref[:, 1, :] + dv_ref[:, 1, :]
    u2 = v_ref[:, 2, :] + dv_ref[:, 2, :]
    wv = wv_ref[...]
    bv = bv_ref[...]
    m0 = jnp.dot(u0, wv, preferred_element_type=jnp.float32) + bv
    m1 = jnp.dot(u1, wv, preferred_element_type=jnp.float32) + bv
    m2 = jnp.dot(u2, wv, preferred_element_type=jnp.float32) + bv
    l0, r0 = m0[:, :H], m0[:, H:]
    l1, r1 = m1[:, :H], m1[:, H:]
    l2, r2 = m2[:, :H], m2[:, H:]
    vnorm = jnp.sqrt(l0 * l0 + l1 * l1 + l2 * l2 + EPS)
    h = (jnp.dot(s1, wm1a_ref[...], preferred_element_type=jnp.float32)
         + jnp.dot(vnorm, wm1b_ref[...], preferred_element_type=jnp.float32)
         + bm1_ref[...])
    h = h * jax.nn.sigmoid(h)
    mix = jnp.dot(h, wm2_ref[...], preferred_element_type=jnp.float32)
    mix = mix + bm2_ref[...]
    ds2 = mix[:, :H]
    dvu = mix[:, H:2 * H]
    dsv = mix[:, 2 * H:]
    dot_lr = l0 * r0 + l1 * r1 + l2 * r2
    so_ref[...] = s1 + ds2 + dsv * dot_lr
    vo_ref[:, 0, :] = u0 + dvu * r0
    vo_ref[:, 1, :] = u1 + dvu * r1
    vo_ref[:, 2, :] = u2 + dvu * r2


def _mixing(s2, v, ds, dv, Wv, bv, Wm1, bm1, Wm2, bm2):
    n = s2.shape[0]
    r = 1000
    grid = n // r
    return pl.pallas_call(
        _mix_body,
        grid=(grid,),
        in_specs=[
            pl.BlockSpec((r, H), lambda i: (i, 0)),
            pl.BlockSpec((r, 3, H), lambda i: (i, 0, 0)),
            pl.BlockSpec((r, H), lambda i: (i, 0)),
            pl.BlockSpec((r, 3, H), lambda i: (i, 0, 0)),
            pl.BlockSpec((H, 2 * H), lambda i: (0, 0)),
            pl.BlockSpec((1, 2 * H), lambda i: (0, 0)),
            pl.BlockSpec((H, H), lambda i: (0, 0)),
            pl.BlockSpec((H, H), lambda i: (0, 0)),
            pl.BlockSpec((1, H), lambda i: (0, 0)),
            pl.BlockSpec((H, 3 * H), lambda i: (0, 0)),
            pl.BlockSpec((1, 3 * H), lambda i: (0, 0)),
        ],
        out_specs=[
            pl.BlockSpec((r, H), lambda i: (i, 0)),
            pl.BlockSpec((r, 3, H), lambda i: (i, 0, 0)),
        ],
        out_shape=[
            jax.ShapeDtypeStruct((n, H), jnp.float32),
            jax.ShapeDtypeStruct((n, 3, H), jnp.float32),
        ],
    )(s2, v, ds, dv, Wv, bv.reshape(1, 2 * H), Wm1[:H], Wm1[H:],
      bm1.reshape(1, H), Wm2, bm2.reshape(1, 3 * H))


def kernel(s, v, dir_ij, Wij, senders, receivers,
           Wi1, bi1, Wi2, bi2, Wv, bv, Wm1, bm1, Wm2, bm2):
    n = s.shape[0]
    e = senders.shape[0]
    s2 = s.reshape(n, H)
    wi2p = Wi2.reshape(H, 3, NCHUNK, CW).transpose(0, 2, 1, 3).reshape(
        H, 3 * H)
    bi2p = bi2.reshape(3, NCHUNK, CW).transpose(1, 0, 2).reshape(3 * H)
    x = _interaction(s2, Wi1, bi1, wi2p, bi2p)
    x8 = x.reshape(n * NCHUNK, 3 * CW)
    v8 = v.reshape(n, 3, NCHUNK, CW).transpose(0, 2, 1, 3).reshape(
        n * NCHUNK, 3 * CW)
    wij = Wij.reshape(e, 3, NCHUNK, CW)
    sr2 = jnp.stack([receivers, senders], axis=0)
    ds8, dv8 = _edge_phase(x8, v8, wij, dir_ij.reshape(e * 3), sr2, n)
    ds = ds8.reshape(n, H)
    dv = dv8.reshape(n, 3, H)
    so, vo = _mixing(s2, v, ds, dv, Wv, bv, Wm1, bm1, Wm2, bm2)
    return (so.reshape(n, 1, H), vo)


# speedup vs baseline: 1.2413x; 1.0023x over previous
import functools

import jax
import jax.numpy as jnp
from jax import lax
from jax.experimental import pallas as pl
from jax.experimental.pallas import tpu as pltpu
from jax.experimental.pallas import tpu_sc as plsc

H = 256
EPS = 1e-08

NC = 2
NS = 16
LANES = 16
NCHUNK = 8
CW = 32
B = 80


def _mlp_a_body(s_ref, w1_ref, b1_ref, w2_ref, b2_ref, o_ref):
    h = jnp.dot(s_ref[...], w1_ref[...], preferred_element_type=jnp.float32)
    h = h + b1_ref[...]
    h = h * jax.nn.sigmoid(h)
    o = jnp.dot(h, w2_ref[...], preferred_element_type=jnp.float32)
    o_ref[...] = o + b2_ref[...]


def _interaction(s2, Wi1, bi1, Wi2, bi2):
    n = s2.shape[0]
    r = 2000
    grid = n // r
    return pl.pallas_call(
        _mlp_a_body,
        grid=(grid,),
        in_specs=[
            pl.BlockSpec((r, H), lambda i: (i, 0)),
            pl.BlockSpec((H, H), lambda i: (0, 0)),
            pl.BlockSpec((1, H), lambda i: (0, 0)),
            pl.BlockSpec((H, 3 * H), lambda i: (0, 0)),
            pl.BlockSpec((1, 3 * H), lambda i: (0, 0)),
        ],
        out_specs=pl.BlockSpec((r, 3 * H), lambda i: (i, 0)),
        out_shape=jax.ShapeDtypeStruct((n, 3 * H), jnp.float32),
    )(s2, Wi1, bi1.reshape(1, H), Wi2, bi2.reshape(1, 3 * H))


def _edge_phase(x8, v8, wij, dir_flat, sr2, n):
    e = sr2.shape[1]
    nblk_per_tile = e // B // NS
    npair = (nblk_per_tile - 1) // 2
    rows_per_tile = n // NS

    mesh = plsc.VectorSubcoreMesh(
        core_axis_name="c", subcore_axis_name="s",
        num_cores=NC, num_subcores=NS)

    DEPTH = 2
    buf_set = [
        pltpu.VMEM((2, B), jnp.int32),
        pltpu.VMEM((B,), jnp.int32),
        pltpu.VMEM((B, 3 * CW), jnp.float32),
        pltpu.VMEM((B, 3 * CW), jnp.float32),
        pltpu.VMEM((B * 3 + LANES,), jnp.float32),
    ]

    @functools.partial(
        pl.kernel,
        out_type=(
            jax.ShapeDtypeStruct((n, NCHUNK, CW), jnp.float32),
            jax.ShapeDtypeStruct((n, 3, NCHUNK, CW), jnp.float32),
        ),
        mesh=mesh,
        scratch_types=[
            pltpu.VMEM_SHARED((n, 4 * CW), jnp.float32),
            pltpu.VMEM((5, 4 * CW), jnp.float32),
            pltpu.VMEM((B, 3, CW), jnp.float32),
            pltpu.VMEM((B, 4 * CW), jnp.float32),
            pltpu.SemaphoreType.DMA((DEPTH,)),
        ] + buf_set * DEPTH,
        compiler_params=pltpu.CompilerParams(use_tc_tiling_on_sc=False),
    )
    def ek(x_hbm, v_hbm, w_hbm, dir_hbm, sr_hbm, ds_out, dv_out,
           accum, zbuf, wbuf, orows, sems, *bufs):
        cid = lax.axis_index("c")
        tid = lax.axis_index("s")
        nb = len(buf_set)
        pbufs = tuple(bufs[i * nb:(i + 1) * nb] for i in range(DEPTH))

        nz = (4 * CW) // LANES
        def zfill(i, _):
            zbuf[i // nz, pl.ds((i % nz) * LANES, LANES)] = jnp.zeros(
                (LANES,), jnp.float32)
            return 0
        lax.fori_loop(0, 5 * nz, zfill, 0)

        n0 = tid * rows_per_tile

        for ch_l in range(NCHUNK // NC):
            ch = cid * (NCHUNK // NC) + ch_l

            for z in range(rows_per_tile // 5):
                pltpu.sync_copy(
                    zbuf, accum.at[pl.ds(n0 + z * 5, 5), :])
            plsc.subcore_barrier()

            def issue(bi, p):
                (srb, gx, xblk, vblk, dirb) = pbufs[p]
                sem = sems.at[p]
                e0 = (bi * NS + tid) * B
                pltpu.sync_copy(sr_hbm.at[:, pl.ds(e0, B)], srb)

                def idx_body(k, _):
                    sl = pl.ds(k * LANES, LANES)
                    gx[sl] = srb[0, sl] * NCHUNK + ch
                    return 0
                lax.fori_loop(0, B // LANES, idx_body, 0)

                return (
                    pltpu.async_copy(x_hbm.at[gx], xblk, sem),
                    pltpu.async_copy(v_hbm.at[gx], vblk, sem),
                    pltpu.async_copy(dir_hbm.at[pl.ds(e0 * 3, B * 3)],
                                     dirb.at[pl.ds(0, B * 3)], sem),
                )

            def compute(descs, bi, p):
                (srb, gx, xblk, vblk, dirb) = pbufs[p]
                e0 = (bi * NS + tid) * B
                pltpu.sync_copy(w_hbm.at[pl.ds(e0, B), :, ch, :], wbuf)
                for d in descs:
                    d.wait()

                def e_body(ei, _):
                    d3 = dirb[pl.ds(ei * 3, LANES)]
                    dd0 = d3[0]
                    dd1 = d3[1]
                    dd2 = d3[2]
                    for j in range(CW // LANES):
                        sl = pl.ds(j * LANES, LANES)
                        sl1 = pl.ds(CW + j * LANES, LANES)
                        sl2 = pl.ds(2 * CW + j * LANES, LANES)
                        a1 = xblk[ei, sl1] * wbuf[ei, 1, sl]
                        a2 = xblk[ei, sl2] * wbuf[ei, 2, sl]
                        orows[ei, pl.ds(j * LANES, LANES)] = (
                            xblk[ei, sl] * wbuf[ei, 0, sl])
                        orows[ei, pl.ds(CW + j * LANES, LANES)] = (
                            a1 * dd0 + a2 * vblk[ei, sl])
                        orows[ei, pl.ds(2 * CW + j * LANES, LANES)] = (
                            a1 * dd1 + a2 * vblk[ei, sl1])
                        orows[ei, pl.ds(3 * CW + j * LANES, LANES)] = (
                            a1 * dd2 + a2 * vblk[ei, sl2])
                    return 0
                lax.fori_loop(0, B, e_body, 0, unroll=2)

                pltpu.sync_copy(orows, accum.at[srb.at[1]], add=True)

            def pair_body(k, _):
                d0 = issue(2 * k, 0)
                d1 = issue(2 * k + 1, 1)
                compute(d0, 2 * k, 0)
                compute(d1, 2 * k + 1, 1)
                return 0
            lax.fori_loop(0, npair, pair_body, 0)
            dt = issue(jnp.int32(nblk_per_tile - 1), 0)
            compute(dt, jnp.int32(nblk_per_tile - 1), 0)
            plsc.subcore_barrier()

            pltpu.sync_copy(
                accum.at[pl.ds(n0, rows_per_tile), pl.ds(0, CW)],
                ds_out.at[pl.ds(n0, rows_per_tile), ch, :])
            for d in range(3):
                pltpu.sync_copy(
                    accum.at[pl.ds(n0, rows_per_tile),
                             pl.ds((d + 1) * CW, CW)],
                    dv_out.at[pl.ds(n0, rows_per_tile), d, ch, :])

    return ek(x8, v8, wij, dir_flat, sr2)


def _mix_body(s_ref, v_ref, ds_ref, dv_ref, wv_ref, bv_ref,
              wm1a_ref, wm1b_ref, bm1_ref, wm2_ref, bm2_ref,
              so_ref, vo_ref):
    s1 = s_ref[...] + ds_ref[...]
    u0 = v_ref[:, 0, :] + dv_ref[:, 0, :]
    u1 = v_ref[:, 1, :] + dv_ref[:, 1, :]
    u2 = v_ref[:, 2, :] + dv_ref[:, 2, :]
    wv = wv_ref[...]
    bv = bv_ref[...]
    m0 = jnp.dot(u0, wv, preferred_element_type=jnp.float32) + bv
    m1 = jnp.dot(u1, wv, preferred_element_type=jnp.float32) + bv
    m2 = jnp.dot(u2, wv, preferred_element_type=jnp.float32) + bv
    l0, r0 = m0[:, :H], m0[:, H:]
    l1, r1 = m1[:, :H], m1[:, H:]
    l2, r2 = m2[:, :H], m2[:, H:]
    vnorm = jnp.sqrt(l0 * l0 + l1 * l1 + l2 * l2 + EPS)
    h = (jnp.dot(s1, wm1a_ref[...], preferred_element_type=jnp.float32)
         + jnp.dot(vnorm, wm1b_ref[...], preferred_element_type=jnp.float32)
         + bm1_ref[...])
    h = h * jax.nn.sigmoid(h)
    mix = jnp.dot(h, wm2_ref[...], preferred_element_type=jnp.float32)
    mix = mix + bm2_ref[...]
    ds2 = mix[:, :H]
    dvu = mix[:, H:2 * H]
    dsv = mix[:, 2 * H:]
    dot_lr = l0 * r0 + l1 * r1 + l2 * r2
    so_ref[...] = s1 + ds2 + dsv * dot_lr
    vo_ref[:, 0, :] = u0 + dvu * r0
    vo_ref[:, 1, :] = u1 + dvu * r1
    vo_ref[:, 2, :] = u2 + dvu * r2


def _mixing(s2, v, ds, dv, Wv, bv, Wm1, bm1, Wm2, bm2):
    n = s2.shape[0]
    r = 1000
    grid = n // r
    return pl.pallas_call(
        _mix_body,
        grid=(grid,),
        in_specs=[
            pl.BlockSpec((r, H), lambda i: (i, 0)),
            pl.BlockSpec((r, 3, H), lambda i: (i, 0, 0)),
            pl.BlockSpec((r, H), lambda i: (i, 0)),
            pl.BlockSpec((r, 3, H), lambda i: (i, 0, 0)),
            pl.BlockSpec((H, 2 * H), lambda i: (0, 0)),
            pl.BlockSpec((1, 2 * H), lambda i: (0, 0)),
            pl.BlockSpec((H, H), lambda i: (0, 0)),
            pl.BlockSpec((H, H), lambda i: (0, 0)),
            pl.BlockSpec((1, H), lambda i: (0, 0)),
            pl.BlockSpec((H, 3 * H), lambda i: (0, 0)),
            pl.BlockSpec((1, 3 * H), lambda i: (0, 0)),
        ],
        out_specs=[
            pl.BlockSpec((r, H), lambda i: (i, 0)),
            pl.BlockSpec((r, 3, H), lambda i: (i, 0, 0)),
        ],
        out_shape=[
            jax.ShapeDtypeStruct((n, H), jnp.float32),
            jax.ShapeDtypeStruct((n, 3, H), jnp.float32),
        ],
    )(s2, v, ds, dv, Wv, bv.reshape(1, 2 * H), Wm1[:H], Wm1[H:],
      bm1.reshape(1, H), Wm2, bm2.reshape(1, 3 * H))


def kernel(s, v, dir_ij, Wij, senders, receivers,
           Wi1, bi1, Wi2, bi2, Wv, bv, Wm1, bm1, Wm2, bm2):
    n = s.shape[0]
    e = senders.shape[0]
    s2 = s.reshape(n, H)
    wi2p = Wi2.reshape(H, 3, NCHUNK, CW).transpose(0, 2, 1, 3).reshape(
        H, 3 * H)
    bi2p = bi2.reshape(3, NCHUNK, CW).transpose(1, 0, 2).reshape(3 * H)
    x = _interaction(s2, Wi1, bi1, wi2p, bi2p)
    x8 = x.reshape(n * NCHUNK, 3 * CW)
    v8 = v.reshape(n, 3, NCHUNK, CW).transpose(0, 2, 1, 3).reshape(
        n * NCHUNK, 3 * CW)
    wij = Wij.reshape(e, 3, NCHUNK, CW)
    sr2 = jnp.stack([receivers, senders], axis=0)
    ds8, dv8 = _edge_phase(x8, v8, wij, dir_ij.reshape(e * 3), sr2, n)
    ds = ds8.reshape(n, H)
    dv = dv8.reshape(n, 3, H)
    so, vo = _mixing(s2, v, ds, dv, Wv, bv, Wm1, bm1, Wm2, bm2)
    return (so.reshape(n, 1, H), vo)
